# Initial kernel scaffold; baseline (speedup 1.0000x reference)
import jax
import jax.numpy as jnp
from jax.experimental import pallas as pl


def kernel(x, W_cross, W_linear, bias):
    raise NotImplementedError("write your pallas kernel here")


# speedup vs baseline: 23.5304x; 23.5304x over previous
import functools

import jax
import jax.numpy as jnp
from jax import lax
from jax.experimental import pallas as pl
from jax.experimental.pallas import tpu as pltpu
from jax.experimental.pallas import tpu_sc as plsc

F = 26
VOCAB = 1000
V = F * VOCAB
D = 16
B = 4096
NPAIR = F * (F - 1) // 2
NC, NS = 2, 16
NW = NC * NS
CHUNK = B // NW
NROWS = NPAIR + 3
VPAD = 26112


def _sc_body(xt_hbm, wf_hbm, wl_hbm, part_hbm,
             taba, tabb, xa, xb, acc, wl_v, xck, lout, zbuf):
    wid = lax.axis_index("s") * NC + lax.axis_index("c")

    pltpu.sync_copy(wl_hbm, wl_v)
    c0 = wid * CHUNK
    for f in range(F):
        pltpu.sync_copy(xt_hbm.at[pl.ds(f * B + c0, CHUNK)],
                        xck.at[pl.ds(f * CHUNK, CHUNK)])

    def lin_chunk(c, carry):
        s = c * 16
        lv = jnp.zeros((16,), jnp.float32)
        for f in range(F):
            xi = xck[pl.ds(f * CHUNK + s, 16)]
            lv = lv + plsc.load_gather(wl_v, [xi + f * VOCAB])
        lout[pl.ds(s, 16)] = lv
        zbuf[pl.ds(s, 16)] = jnp.zeros((16,), jnp.float32)
        return carry

    lax.fori_loop(0, CHUNK // 16, lin_chunk, 0)
    pltpu.sync_copy(lout, part_hbm.at[pl.ds(NPAIR * B + c0, CHUNK)])
    pltpu.sync_copy(zbuf, part_hbm.at[pl.ds((NPAIR + 1) * B + c0, CHUNK)])
    pltpu.sync_copy(zbuf, part_hbm.at[pl.ds((NPAIR + 2) * B + c0, CHUNK)])

    start = 10 * wid + jnp.minimum(wid, 5)
    cnt = jnp.where(wid < 5, 11, 10)

    def task(p, carry):
        r = lax.shift_right_logical(p * 5042, 16)
        g = p - 13 * r
        t = r + g
        u = r + 25 - g
        i = jnp.where(g == 0, r, jnp.where(t >= 25, t - 25, t))
        j = jnp.where(g == 0, 25, jnp.where(u >= 25, u - 25, u))
        pltpu.sync_copy(xt_hbm.at[pl.ds(i * B, B)], xa)
        pltpu.sync_copy(xt_hbm.at[pl.ds(j * B, B)], xb)
        pltpu.sync_copy(wf_hbm.at[pl.ds((j * V + i * VOCAB) * D, VOCAB * D)], taba)
        pltpu.sync_copy(wf_hbm.at[pl.ds((i * V + j * VOCAB) * D, VOCAB * D)], tabb)

        def chunk(c, inner):
            s = c * 16
            ba = xa[pl.ds(s, 16)] * D
            bb = xb[pl.ds(s, 16)] * D
            accv = jnp.zeros((16,), jnp.float32)
            for d in range(D):
                a = plsc.load_gather(taba, [ba + d])
                b = plsc.load_gather(tabb, [bb + d])
                accv = accv + a * b
            acc[pl.ds(s, 16)] = accv
            return inner

        lax.fori_loop(0, B // 16, chunk, 0)
        pltpu.sync_copy(acc, part_hbm.at[pl.ds(p * B, B)])
        return carry

    lax.fori_loop(start, start + cnt, task, 0)


_sc_stage = functools.partial(
    pl.kernel,
    out_type=jax.ShapeDtypeStruct((NROWS * B,), jnp.float32),
    mesh=plsc.VectorSubcoreMesh(core_axis_name="c", subcore_axis_name="s"),
    compiler_params=pltpu.CompilerParams(needs_layout_passes=False),
    scratch_types=[
        pltpu.VMEM((VOCAB * D,), jnp.float32),
        pltpu.VMEM((VOCAB * D,), jnp.float32),
        pltpu.VMEM((B,), jnp.int32),
        pltpu.VMEM((B,), jnp.int32),
        pltpu.VMEM((B,), jnp.float32),
        pltpu.VMEM((VPAD,), jnp.float32),
        pltpu.VMEM((F * CHUNK,), jnp.int32),
        pltpu.VMEM((CHUNK,), jnp.float32),
        pltpu.VMEM((CHUNK,), jnp.float32),
    ],
)(_sc_body)


def _combine_body(part_ref, bias_ref, o_ref):
    z = jnp.sum(part_ref[...], axis=0) + bias_ref[0]
    o_ref[...] = jax.nn.sigmoid(z)


_combine = pl.pallas_call(
    _combine_body,
    out_shape=jax.ShapeDtypeStruct((B,), jnp.float32),
    in_specs=[
        pl.BlockSpec(memory_space=pltpu.VMEM),
        pl.BlockSpec(memory_space=pltpu.SMEM),
    ],
    out_specs=pl.BlockSpec(memory_space=pltpu.VMEM),
)


def kernel(x, W_cross, W_linear, bias):
    xt = x.T.reshape(-1)
    wf = W_cross.reshape(-1)
    wl = jnp.pad(W_linear.reshape(-1), (0, VPAD - V))
    part = _sc_stage(xt, wf, wl)
    out = _combine(part.reshape(NROWS, B), bias)
    return out.reshape(B, 1)
